# baseline (speedup 1.0000x reference)
import dataclasses
import functools

import jax
import jax.numpy as jnp
from jax import lax
from jax.experimental import pallas as pl
from jax.experimental.pallas import tpu as pltpu
from jax.experimental.pallas import tpu_sc as plsc

N = 10000
E = 320000
D = 128
OUT = 128

NC = 2
NS = 16
NW = NC * NS
CH = 80
CPW = 128
NCHUNKS = NW * CPW
E_PAD = NCHUNKS * CH
NBUF = 4
NIDX = 8
STEP = 8
ROWS_PER_SUB = 624
TAIL_ROWS = N - NS * ROWS_PER_SUB
LANES = 16
GRP = CH // LANES


def _sc_aggregate(x, src2, dst2, wt2, zeros):
    mesh = plsc.VectorSubcoreMesh(core_axis_name="c", subcore_axis_name="s")
    cp = pltpu.CompilerParams()
    if "needs_layout_passes" in pltpu.CompilerParams.__dataclass_fields__:
        cp = dataclasses.replace(cp, needs_layout_passes=False)

    @functools.partial(
        pl.kernel,
        out_type=jax.ShapeDtypeStruct((NC, N, D), jnp.float32),
        mesh=mesh,
        compiler_params=cp,
        scratch_types=[
            pltpu.VMEM((NIDX, CH), jnp.int32),
            pltpu.VMEM((NIDX, CH), jnp.int32),
            pltpu.VMEM((NIDX, CH), jnp.float32),
            pltpu.VMEM((NBUF, CH, D), jnp.float32),
            pltpu.VMEM_SHARED((N, D), jnp.float32),
        ] + [pltpu.SemaphoreType.DMA] * (2 * NBUF + NIDX),
    )
    def agg(x_hbm, src_hbm, dst_hbm, wt_hbm, z_hbm, hp_hbm,
            srcr, dstr, wtr, rows_v, h_sh, *sems):
        gsems = sems[0:NBUF]
        ssems = sems[NBUF:2 * NBUF]
        isems = sems[2 * NBUF:]
        cid = lax.axis_index("c")
        sid = lax.axis_index("s")
        wid = sid * NC + cid
        row0 = sid * ROWS_PER_SUB
        base = wid * CPW

        pltpu.sync_copy(z_hbm.at[pl.ds(row0, ROWS_PER_SUB)],
                        h_sh.at[pl.ds(row0, ROWS_PER_SUB)])

        @pl.when(sid == NS - 1)
        def _zero_tail():
            pltpu.sync_copy(z_hbm.at[pl.ds(NS * ROWS_PER_SUB, TAIL_ROWS)],
                            h_sh.at[pl.ds(NS * ROWS_PER_SUB, TAIL_ROWS)])

        plsc.subcore_barrier()

        def issue_idx(l, s):
            pltpu.async_copy(src_hbm.at[base + l], srcr.at[s], isems[s])
            pltpu.async_copy(dst_hbm.at[base + l], dstr.at[s], isems[s])
            pltpu.async_copy(wt_hbm.at[base + l], wtr.at[s], isems[s])

        def wait_idx(l, s):
            pltpu.make_async_copy(src_hbm.at[base + l], srcr.at[s],
                                  isems[s]).wait()
            pltpu.make_async_copy(dst_hbm.at[base + l], dstr.at[s],
                                  isems[s]).wait()
            pltpu.make_async_copy(wt_hbm.at[base + l], wtr.at[s],
                                  isems[s]).wait()

        def issue_gather(s, b):
            pltpu.async_copy(x_hbm.at[srcr.at[s]], rows_v.at[b], gsems[b])

        def wait_gather(s, b):
            pltpu.make_async_copy(x_hbm.at[srcr.at[s]], rows_v.at[b],
                                  gsems[b]).wait()

        def issue_scatter(s, b):
            pltpu.async_copy(rows_v.at[b], h_sh.at[dstr.at[s]], ssems[b],
                             add=True)

        def drain_scatter(s, b):
            pltpu.make_async_copy(rows_v.at[b], h_sh.at[dstr.at[s]],
                                  ssems[b]).wait()

        def scale_rows(b, s):
            buf = rows_v.at[b]
            sidx = jnp.full((LANES,), s, jnp.int32)

            @pl.loop(0, GRP)
            def _grp(g):
                for jj in range(LANES):
                    e = g * LANES + jj
                    w = plsc.load_gather(
                        wtr, [sidx, jnp.full((LANES,), e, jnp.int32)])
                    for dd in range(D // LANES):
                        sl = pl.ds(dd * LANES, LANES)
                        buf[e, sl] = buf[e, sl] * w

        for k in range(NIDX - 2):
            issue_idx(k, k)
        for k in range(2):
            wait_idx(k, k)
            issue_gather(k, k)

        @pl.loop(0, CPW, step=STEP)
        def _octet(c):
            for b8 in range(STEP):
                l = c + b8
                b = b8 % NBUF

                @pl.when(l + 2 < CPW)
                def _prefetch():
                    @pl.when(l >= 2)
                    def _drain():
                        drain_scatter((b8 - 2) % NIDX, (b8 + 2) % NBUF)

                    wait_idx(l + 2, (b8 + 2) % NIDX)
                    issue_gather((b8 + 2) % NIDX, (b8 + 2) % NBUF)

                @pl.when(l + NIDX - 2 < CPW)
                def _idx_ahead():
                    issue_idx(l + NIDX - 2, (b8 - 2) % NIDX)

                wait_gather(b8, b)
                scale_rows(b, b8)
                issue_scatter(b8, b)

        for k in range(CPW - NBUF, CPW):
            drain_scatter(k % NIDX, k % NBUF)

        plsc.subcore_barrier()
        pltpu.sync_copy(h_sh.at[pl.ds(row0, ROWS_PER_SUB)],
                        hp_hbm.at[cid, pl.ds(row0, ROWS_PER_SUB)])

        @pl.when(sid == NS - 1)
        def _flush_tail():
            pltpu.sync_copy(h_sh.at[pl.ds(NS * ROWS_PER_SUB, TAIL_ROWS)],
                            hp_hbm.at[cid, pl.ds(NS * ROWS_PER_SUB, TAIL_ROWS)])

    return agg(x, src2, dst2, wt2, zeros)


def _tc_project(h0, h1, x, wt, wb):
    RB = 1000

    def body(h0_ref, h1_ref, x_ref, wt_ref, wb_ref, o_ref):
        h = h0_ref[...] + h1_ref[...]
        acc = jnp.dot(h, wt_ref[...], preferred_element_type=jnp.float32)
        acc = acc + jnp.dot(x_ref[...], wb_ref[...],
                            preferred_element_type=jnp.float32)
        o_ref[...] = jnp.maximum(acc, 0.0)

    return pl.pallas_call(
        body,
        grid=(N // RB,),
        in_specs=[
            pl.BlockSpec((RB, D), lambda i: (i, 0)),
            pl.BlockSpec((RB, D), lambda i: (i, 0)),
            pl.BlockSpec((RB, D), lambda i: (i, 0)),
            pl.BlockSpec((D, OUT), lambda i: (0, 0)),
            pl.BlockSpec((D, OUT), lambda i: (0, 0)),
        ],
        out_specs=pl.BlockSpec((RB, OUT), lambda i: (i, 0)),
        out_shape=jax.ShapeDtypeStruct((N, OUT), jnp.float32),
    )(h0, h1, x, wt, wb)


def kernel(x, edge_index, edge_weight, W):
    pad = E_PAD - E
    src2 = jnp.concatenate(
        [edge_index[1], jnp.zeros((pad,), jnp.int32)]).reshape(NCHUNKS, CH)
    dst2 = jnp.concatenate(
        [edge_index[0], jnp.zeros((pad,), jnp.int32)]).reshape(NCHUNKS, CH)
    wt2 = jnp.concatenate(
        [edge_weight, jnp.zeros((pad,), jnp.float32)]).reshape(NCHUNKS, CH)
    zeros = jnp.zeros((N, D), jnp.float32)
    hp = _sc_aggregate(x, src2, dst2, wt2, zeros)
    return _tc_project(hp[0], hp[1], x, W[:D], W[D:])
